# baseline (speedup 1.0000x reference)
import functools

import jax
import jax.numpy as jnp
from jax import lax
from jax.experimental import pallas as pl
from jax.experimental.pallas import tpu as pltpu
from jax.experimental.pallas import tpu_sc as plsc

VOCAB = 1000000
EMBED_DIM = 64
BATCH = 16384
HIST = 50

_INFO = plsc.get_sparse_core_info()
_NC = _INFO.num_cores
_NS = _INFO.num_subcores
_NW = _NC * _NS

_BPW = BATCH // _NW
_NBUF = 8


def _make_gather():
    mesh = plsc.VectorSubcoreMesh(core_axis_name="c", subcore_axis_name="s")

    @functools.partial(
        pl.kernel,
        mesh=mesh,
        compiler_params=pltpu.CompilerParams(use_tc_tiling_on_sc=False),
        out_type=jax.ShapeDtypeStruct((BATCH, HIST, EMBED_DIM), jnp.float32),
        scratch_types=[
            pltpu.VMEM((_BPW, HIST), jnp.int32),
            [pltpu.VMEM((HIST, EMBED_DIM), jnp.float32)] * _NBUF,
            [pltpu.SemaphoreType.DMA] * _NBUF,
            [pltpu.SemaphoreType.DMA] * _NBUF,
        ],
    )
    def gather_kernel(idx_hbm, table_hbm, out_hbm, idx_v, bufs, gsems, wsems):
        wid = lax.axis_index("s") * _NC + lax.axis_index("c")
        base = wid * _BPW
        pltpu.sync_copy(idx_hbm.at[pl.ds(base, _BPW)], idx_v)

        def start_gather(j, b):
            pltpu.async_copy(table_hbm.at[idx_v.at[j]], bufs[b], gsems[b])

        def finish_gather(j, b):
            pltpu.make_async_copy(table_hbm.at[idx_v.at[j]], bufs[b], gsems[b]).wait()

        def start_write(j, b):
            pltpu.async_copy(bufs[b], out_hbm.at[base + j], wsems[b])

        def finish_write(j, b):
            pltpu.make_async_copy(bufs[b], out_hbm.at[base + j], wsems[b]).wait()

        def body(g, carry):
            for b in range(_NBUF):
                j = g * _NBUF + b
                @pl.when(g > 0)
                def _(b=b, j=j):
                    finish_write(j - _NBUF, b)

                start_gather(j, b)

                kb = (b + 1) % _NBUF
                if b == _NBUF - 1:
                    finish_gather(j - _NBUF + 1, kb)
                    start_write(j - _NBUF + 1, kb)
                else:
                    @pl.when(g > 0)
                    def _(b=b, j=j, kb=kb):
                        finish_gather(j - _NBUF + 1, kb)
                        start_write(j - _NBUF + 1, kb)
            return carry

        lax.fori_loop(0, _BPW // _NBUF, body, 0, unroll=False)

        for i in range(1, _NBUF):
            k = _BPW - _NBUF + i
            finish_gather(k, i)
            start_write(k, i)
        for b in range(_NBUF):
            finish_write(_BPW - _NBUF + b, b)

    return gather_kernel


_gather = _make_gather()


def kernel(X, table):
    return _gather(X.astype(jnp.int32), table)


# speedup vs baseline: 1.1104x; 1.1104x over previous
import functools

import jax
import jax.numpy as jnp
from jax import lax
from jax.experimental import pallas as pl
from jax.experimental.pallas import tpu as pltpu
from jax.experimental.pallas import tpu_sc as plsc

VOCAB = 1000000
EMBED_DIM = 64
BATCH = 16384
HIST = 50

_INFO = plsc.get_sparse_core_info()
_NC = _INFO.num_cores
_NS = _INFO.num_subcores
_NW = _NC * _NS

_BPW = BATCH // _NW
_CSZ = 128
_SPH = _BPW // _CSZ
_CH = HIST * _SPH
_NBUF = 8


def _make_gather():
    mesh = plsc.VectorSubcoreMesh(core_axis_name="c", subcore_axis_name="s")

    @functools.partial(
        pl.kernel,
        mesh=mesh,
        compiler_params=pltpu.CompilerParams(use_tc_tiling_on_sc=False),
        out_type=jax.ShapeDtypeStruct((HIST, BATCH, EMBED_DIM), jnp.float32),
        scratch_types=[
            pltpu.VMEM((HIST, _BPW), jnp.int32),
            [pltpu.VMEM((_CSZ, EMBED_DIM), jnp.float32)] * _NBUF,
            [pltpu.SemaphoreType.DMA] * _NBUF,
            [pltpu.SemaphoreType.DMA] * _NBUF,
        ],
    )
    def gather_kernel(idx_hbm, table_hbm, out_hbm, idx_v, bufs, gsems, wsems):
        wid = lax.axis_index("s") * _NC + lax.axis_index("c")
        base = wid * _BPW
        pltpu.sync_copy(idx_hbm.at[:, pl.ds(base, _BPW)], idx_v)

        def idx_at(c):
            h = c // _SPH
            s = c % _SPH
            return idx_v.at[h, pl.ds(s * _CSZ, _CSZ)]

        def out_at(c):
            h = c // _SPH
            s = c % _SPH
            return out_hbm.at[h, pl.ds(base + s * _CSZ, _CSZ)]

        def start_gather(c, b):
            pltpu.async_copy(table_hbm.at[idx_at(c)], bufs[b], gsems[b])

        def finish_gather(c, b):
            pltpu.make_async_copy(table_hbm.at[idx_at(c)], bufs[b], gsems[b]).wait()

        def start_write(c, b):
            pltpu.async_copy(bufs[b], out_at(c), wsems[b])

        def finish_write(c, b):
            pltpu.make_async_copy(bufs[b], out_at(c), wsems[b]).wait()

        def body(g, carry):
            for b in range(_NBUF):
                c = g * _NBUF + b
                @pl.when(g > 0)
                def _(b=b, c=c):
                    finish_write(c - _NBUF, b)

                start_gather(c, b)

                kb = (b + 1) % _NBUF
                if b == _NBUF - 1:
                    finish_gather(c - _NBUF + 1, kb)
                    start_write(c - _NBUF + 1, kb)
                else:
                    @pl.when(g > 0)
                    def _(b=b, c=c, kb=kb):
                        finish_gather(c - _NBUF + 1, kb)
                        start_write(c - _NBUF + 1, kb)
            return carry

        lax.fori_loop(0, _CH // _NBUF, body, 0, unroll=False)

        for i in range(1, _NBUF):
            k = _CH - _NBUF + i
            finish_gather(k, i)
            start_write(k, i)
        for b in range(_NBUF):
            finish_write(_CH - _NBUF + b, b)

    return gather_kernel


_gather = _make_gather()


def kernel(X, table):
    tab2 = jnp.pad(table, ((0, 0), (0, EMBED_DIM))).reshape(2 * VOCAB, EMBED_DIM)
    xt = X.T.astype(jnp.int32) * 2
    out_t = _gather(xt, tab2)
    return jnp.transpose(out_t, (1, 0, 2))
